# baseline (speedup 1.0000x reference)
import functools

import jax
import jax.numpy as jnp
from jax import lax
from jax.experimental import pallas as pl
from jax.experimental.pallas import tpu as pltpu
from jax.experimental.pallas import tpu_sc as plsc

N = 10000
E = 320000
EL = 100000
DIM = 128

NC = 2
NS = 16
NT = NC * NS

NP = 10240
RPT = NP // NS
CH = 128
CE = 80
EP = NT * CE * CH
CD = 26
ELP = NT * CD * CH
ELT = CD * CH

_mesh = plsc.VectorSubcoreMesh(core_axis_name="c", subcore_axis_name="s")
_f32 = jnp.float32


def _fill16(buf, nrow, val):
    v = jnp.full((16,), val, _f32)

    def body(r, _):
        buf[r, :] = v
        return 0

    lax.fori_loop(0, nrow, body, 0)


def _zero_row_of(idx2d, row):
    z = jnp.zeros((16,), jnp.int32)
    for j in range(CH // 16):
        idx2d[row, pl.ds(j * 16, 16)] = z


@functools.partial(
    pl.kernel,
    out_type=jax.ShapeDtypeStruct((2 * NP, 16), _f32),
    mesh=_mesh,
    scratch_types=[
        pltpu.VMEM((CE, CH), jnp.int32),
        pltpu.VMEM((CH, 16), _f32),
        pltpu.VMEM_SHARED((NP, 16), _f32),
        pltpu.SemaphoreType.DMA,
    ],
)
def _sc_count(dstp3, out, didx, buf, cnt, sem):
    c = lax.axis_index("c")
    s = lax.axis_index("s")
    tile = c * NS + s
    pltpu.sync_copy(dstp3.at[tile], didx)
    _fill16(buf, CH, 0.0)

    def zi(k, _):
        pltpu.sync_copy(buf, cnt.at[pl.ds(s * RPT + k * CH, CH)])
        return 0

    lax.fori_loop(0, RPT // CH, zi, 0)
    _fill16(buf, CH, 1.0)
    plsc.subcore_barrier()

    def grp(g, _):
        def fire(j, _2):
            pltpu.async_copy(buf, cnt.at[didx.at[g * 8 + j]], sem, add=True)
            return 0

        lax.fori_loop(0, 8, fire, 0)

        def drain(j, _2):
            pltpu.make_async_copy(buf, cnt.at[didx.at[g * 8 + j]], sem).wait()
            return 0

        lax.fori_loop(0, 8, drain, 0)
        return 0

    lax.fori_loop(0, CE // 8, grp, 0)
    plsc.subcore_barrier()

    def co(k, _):
        r0 = s * RPT + k * CH
        pltpu.sync_copy(cnt.at[pl.ds(r0, CH)], buf)
        pltpu.sync_copy(buf, out.at[pl.ds(c * NP + r0, CH)])
        return 0

    lax.fori_loop(0, RPT // CH, co, 0)


CE2 = CE // 2


@functools.partial(
    pl.kernel,
    out_type=jax.ShapeDtypeStruct((2 * NP, DIM), _f32),
    mesh=_mesh,
    scratch_types=[
        pltpu.VMEM((CE2 + 1, CH), jnp.int32),
        pltpu.VMEM((CE2, CH), jnp.int32),
        pltpu.VMEM((CH, DIM), _f32),
        pltpu.VMEM((CH, DIM), _f32),
        pltpu.VMEM_SHARED((NP, DIM), _f32),
        pltpu.SemaphoreType.DMA,
        pltpu.SemaphoreType.DMA,
    ],
)
def _sc_agg(table, srcp3, dstp3, out, sidx, didx, rows0, rows1, acc, sem0,
            sem1):
    c = lax.axis_index("c")
    s = lax.axis_index("s")
    tile = c * NS + s
    _zero_row_of(sidx, CE2)

    def zr(r, _):
        for j in range(DIM // 16):
            rows0[r, pl.ds(j * 16, 16)] = jnp.zeros((16,), _f32)
        return 0

    lax.fori_loop(0, CH, zr, 0)

    def zi(k, _):
        pltpu.sync_copy(rows0, acc.at[pl.ds(s * RPT + k * CH, CH)])
        return 0

    lax.fori_loop(0, RPT // CH, zi, 0)
    plsc.subcore_barrier()

    bufs = ((rows0, sem0), (rows1, sem1))

    def phase(p, _):
        pltpu.sync_copy(srcp3.at[tile, pl.ds(p * CE2, CE2)],
                        sidx.at[pl.ds(0, CE2)])
        pltpu.sync_copy(dstp3.at[tile, pl.ds(p * CE2, CE2)], didx)
        pltpu.async_copy(table.at[sidx.at[0]], rows0, sem0)

        def pair(i0, _2):
            for b in range(2):
                i = i0 * 2 + b
                rb, sb = bufs[b]
                rn, sn = bufs[1 - b]
                pltpu.async_copy(table.at[sidx.at[i + 1]], rn, sn)
                pltpu.make_async_copy(table.at[sidx.at[i]], rb, sb).wait()
                pltpu.sync_copy(rb, acc.at[didx.at[i]], add=True)
            return 0

        lax.fori_loop(0, CE2 // 2, pair, 0)
        pltpu.make_async_copy(table.at[sidx.at[CE2]], rows0, sem0).wait()
        return 0

    lax.fori_loop(0, 2, phase, 0)
    plsc.subcore_barrier()

    def co(k, _):
        r0 = s * RPT + k * CH
        pltpu.sync_copy(acc.at[pl.ds(r0, CH)], rows0)
        pltpu.sync_copy(rows0, out.at[pl.ds(c * NP + r0, CH)])
        return 0

    lax.fori_loop(0, RPT // CH, co, 0)


@functools.partial(
    pl.kernel,
    out_type=jax.ShapeDtypeStruct((ELP, 16), _f32),
    mesh=_mesh,
    scratch_types=[
        pltpu.VMEM((CD + 1, CH), jnp.int32),
        pltpu.VMEM((CD + 1, CH), jnp.int32),
        pltpu.VMEM((CH, DIM), _f32),
        pltpu.VMEM((CH, DIM), _f32),
        pltpu.VMEM((CH, DIM), _f32),
        pltpu.VMEM((CH, DIM), _f32),
        pltpu.VMEM((CH, 16), _f32),
        pltpu.SemaphoreType.DMA,
        pltpu.SemaphoreType.DMA,
        pltpu.SemaphoreType.DMA,
        pltpu.SemaphoreType.DMA,
    ],
)
def _sc_decode(z, eli03, eli13, out, idx0, idx1, r0a, r0b, r1a, r1b, rbuf,
               s0a, s0b, s1a, s1b):
    c = lax.axis_index("c")
    s = lax.axis_index("s")
    tile = c * NS + s
    pltpu.sync_copy(eli03.at[tile], idx0.at[pl.ds(0, CD)])
    pltpu.sync_copy(eli13.at[tile], idx1.at[pl.ds(0, CD)])
    _zero_row_of(idx0, CD)
    _zero_row_of(idx1, CD)

    bufs = (((r0a, s0a), (r1a, s1a)), ((r0b, s0b), (r1b, s1b)))
    pltpu.async_copy(z.at[idx0.at[0]], r0a, s0a)
    pltpu.async_copy(z.at[idx1.at[0]], r1a, s1a)
    base = tile * ELT

    def pair(i0, _):
        for b in range(2):
            i = i0 * 2 + b
            (rb0, sb0), (rb1, sb1) = bufs[b]
            (rn0, sn0), (rn1, sn1) = bufs[1 - b]
            pltpu.async_copy(z.at[idx0.at[i + 1]], rn0, sn0)
            pltpu.async_copy(z.at[idx1.at[i + 1]], rn1, sn1)
            pltpu.make_async_copy(z.at[idx0.at[i]], rb0, sb0).wait()
            pltpu.make_async_copy(z.at[idx1.at[i]], rb1, sb1).wait()

            def edge(e, _2):
                acc = rb0[e, pl.ds(0, 16)] * rb1[e, pl.ds(0, 16)]
                for j in range(1, DIM // 16):
                    acc = acc + (rb0[e, pl.ds(j * 16, 16)]
                                 * rb1[e, pl.ds(j * 16, 16)])
                rbuf[e, :] = acc
                return 0

            lax.fori_loop(0, CH, edge, 0)
            pltpu.sync_copy(rbuf, out.at[pl.ds(base + i * CH, CH)])
        return 0

    lax.fori_loop(0, CD // 2, pair, 0)
    pltpu.make_async_copy(z.at[idx0.at[CD]], r0a, s0a).wait()
    pltpu.make_async_copy(z.at[idx1.at[CD]], r1a, s1a).wait()


_R = 512
_GRID = NP // _R


def _dot(a, b):
    return jnp.dot(a, b, preferred_element_type=_f32,
                   precision=lax.Precision.HIGHEST)


def _tc1_body(x_ref, w_ref, c0_ref, c1_ref, xwp_ref, d2_ref):
    deg = c0_ref[:, 0:1] + c1_ref[:, 0:1] + 1.0
    dinv = 1.0 / jnp.sqrt(deg)
    xwp_ref[...] = dinv * _dot(x_ref[...], w_ref[...])
    d2_ref[...] = jnp.broadcast_to(dinv, d2_ref.shape)


def _tc1(x_pad, W1, cnt2):
    return pl.pallas_call(
        _tc1_body,
        grid=(_GRID,),
        in_specs=[
            pl.BlockSpec((_R, DIM), lambda i: (i, 0)),
            pl.BlockSpec((DIM, DIM), lambda i: (0, 0)),
            pl.BlockSpec((_R, 16), lambda i: (i, 0)),
            pl.BlockSpec((_R, 16), lambda i: (i + _GRID, 0)),
        ],
        out_specs=[
            pl.BlockSpec((_R, DIM), lambda i: (i, 0)),
            pl.BlockSpec((_R, DIM), lambda i: (i, 0)),
        ],
        out_shape=[
            jax.ShapeDtypeStruct((NP, DIM), _f32),
            jax.ShapeDtypeStruct((NP, DIM), _f32),
        ],
    )(x_pad, W1, cnt2, cnt2)


def _tc_mid_body(agg0_ref, agg1_ref, xwp_ref, d2_ref, b_ref, w_ref, out_ref):
    d2 = d2_ref[...]
    pre = d2 * (agg0_ref[...] + agg1_ref[...] + xwp_ref[...]) + b_ref[...]
    h = jnp.maximum(pre, 0.0)
    out_ref[...] = d2 * _dot(h, w_ref[...])


def _tc_mid(agg, xwp, d2, brow, W):
    return pl.pallas_call(
        _tc_mid_body,
        grid=(_GRID,),
        in_specs=[
            pl.BlockSpec((_R, DIM), lambda i: (i, 0)),
            pl.BlockSpec((_R, DIM), lambda i: (i + _GRID, 0)),
            pl.BlockSpec((_R, DIM), lambda i: (i, 0)),
            pl.BlockSpec((_R, DIM), lambda i: (i, 0)),
            pl.BlockSpec((1, DIM), lambda i: (0, 0)),
            pl.BlockSpec((DIM, DIM), lambda i: (0, 0)),
        ],
        out_specs=pl.BlockSpec((_R, DIM), lambda i: (i, 0)),
        out_shape=jax.ShapeDtypeStruct((NP, DIM), _f32),
    )(agg, agg, xwp, d2, brow, W)


def _tc_fin_body(agg0_ref, agg1_ref, xwp_ref, d2_ref, b_ref, out_ref):
    out_ref[...] = (d2_ref[...] * (agg0_ref[...] + agg1_ref[...] + xwp_ref[...])
                    + b_ref[...])


def _tc_fin(agg, xwp, d2, brow):
    return pl.pallas_call(
        _tc_fin_body,
        grid=(_GRID,),
        in_specs=[
            pl.BlockSpec((_R, DIM), lambda i: (i, 0)),
            pl.BlockSpec((_R, DIM), lambda i: (i + _GRID, 0)),
            pl.BlockSpec((_R, DIM), lambda i: (i, 0)),
            pl.BlockSpec((_R, DIM), lambda i: (i, 0)),
            pl.BlockSpec((1, DIM), lambda i: (0, 0)),
        ],
        out_specs=pl.BlockSpec((_R, DIM), lambda i: (i, 0)),
        out_shape=jax.ShapeDtypeStruct((NP, DIM), _f32),
    )(agg, agg, xwp, d2, brow)


_RB = 2048


def _tc_lsum_body(r_ref, out_ref):
    out_ref[...] = jnp.sum(r_ref[...], axis=1)


def _tc_lsum(res16):
    return pl.pallas_call(
        _tc_lsum_body,
        grid=(ELP // _RB,),
        in_specs=[pl.BlockSpec((_RB, 16), lambda i: (i, 0))],
        out_specs=pl.BlockSpec((_RB,), lambda i: (i,)),
        out_shape=jax.ShapeDtypeStruct((ELP,), _f32),
    )(res16)


def kernel(x, edge_index, edge_label_index, W1, b1, W2, b2, W3, b3):
    i32 = jnp.int32
    src = edge_index[0].astype(i32)
    dst = edge_index[1].astype(i32)
    pad_e = jnp.full((EP - E,), N, i32)
    srcp3 = jnp.concatenate([src, pad_e]).reshape(NT, CE, CH)
    dstp3 = jnp.concatenate([dst, pad_e]).reshape(NT, CE, CH)
    pad_l = jnp.zeros((ELP - EL,), i32)
    eli03 = jnp.concatenate([edge_label_index[0].astype(i32),
                             pad_l]).reshape(NT, CD, CH)
    eli13 = jnp.concatenate([edge_label_index[1].astype(i32),
                             pad_l]).reshape(NT, CD, CH)
    x_pad = jnp.concatenate([x, jnp.zeros((NP - N, DIM), _f32)], axis=0)

    cnt2 = _sc_count(dstp3)
    xw1p, d2 = _tc1(x_pad, W1, cnt2)
    agg1 = _sc_agg(xw1p, srcp3, dstp3)
    xw2p = _tc_mid(agg1, xw1p, d2, b1.reshape(1, DIM), W2)
    agg2 = _sc_agg(xw2p, srcp3, dstp3)
    xw3p = _tc_mid(agg2, xw2p, d2, b2.reshape(1, DIM), W3)
    agg3 = _sc_agg(xw3p, srcp3, dstp3)
    z = _tc_fin(agg3, xw3p, d2, b3.reshape(1, DIM))
    res16 = _sc_decode(z, eli03, eli13)
    dots = _tc_lsum(res16)
    return dots[:EL]


# speedup vs baseline: 1.5352x; 1.5352x over previous
import functools

import jax
import jax.numpy as jnp
from jax import lax
from jax.experimental import pallas as pl
from jax.experimental.pallas import tpu as pltpu
from jax.experimental.pallas import tpu_sc as plsc

N = 10000
E = 320000
EL = 100000
DIM = 128

NC = 2
NS = 16
NT = NC * NS

NP = 10240
RPT = NP // NS
CH = 128
EP = 323584
EPT = EP // NT
CE = EPT // CH
ELP = 102400
ELT = ELP // NT
CD = ELT // CH

_mesh = plsc.VectorSubcoreMesh(core_axis_name="c", subcore_axis_name="s")
_f32 = jnp.float32


def _fill16(buf, nrow, val):
    v = jnp.full((16,), val, _f32)

    def body(r, _):
        buf[r, :] = v
        return 0

    lax.fori_loop(0, nrow, body, 0)


@functools.partial(
    pl.kernel,
    out_type=jax.ShapeDtypeStruct((2 * NP, DIM), _f32),
    mesh=_mesh,
    scratch_types=[
        pltpu.VMEM((CH,), jnp.int32),
        pltpu.VMEM((CH, DIM), _f32),
        pltpu.VMEM_SHARED((NP, DIM), _f32),
    ],
)
def _sc_count(dstp, out, didx, rows, acc):
    c = lax.axis_index("c")
    s = lax.axis_index("s")
    tile = c * NS + s

    def zr(r, _):
        def zc(j, _2):
            rows[r, pl.ds(j * 16, 16)] = jnp.zeros((16,), _f32)
            return 0

        lax.fori_loop(0, DIM // 16, zc, 0)
        return 0

    lax.fori_loop(0, CH, zr, 0)

    def zi(k, _):
        pltpu.sync_copy(rows, acc.at[pl.ds(s * RPT + k * CH, CH)])
        return 0

    lax.fori_loop(0, RPT // CH, zi, 0)

    def fr(r, _):
        def fc(j, _2):
            rows[r, pl.ds(j * 16, 16)] = jnp.full((16,), 1.0, _f32)
            return 0

        lax.fori_loop(0, DIM // 16, fc, 0)
        return 0

    lax.fori_loop(0, CH, fr, 0)
    plsc.subcore_barrier()

    base = tile * EPT

    def body(i, _):
        pltpu.sync_copy(dstp.at[pl.ds(base + i * CH, CH)], didx)
        pltpu.sync_copy(rows, acc.at[didx], add=True)
        return 0

    lax.fori_loop(0, CE, body, 0)
    plsc.subcore_barrier()

    def co(k, _):
        r0 = s * RPT + k * CH
        pltpu.sync_copy(acc.at[pl.ds(r0, CH)], rows)
        pltpu.sync_copy(rows, out.at[pl.ds(c * NP + r0, CH)])
        return 0

    lax.fori_loop(0, RPT // CH, co, 0)


@functools.partial(
    pl.kernel,
    out_type=jax.ShapeDtypeStruct((2 * NP, DIM), _f32),
    mesh=_mesh,
    scratch_types=[
        pltpu.VMEM((CH,), jnp.int32),
        pltpu.VMEM((CH,), jnp.int32),
        pltpu.VMEM((CH, DIM), _f32),
        pltpu.VMEM_SHARED((NP, DIM), _f32),
        pltpu.SemaphoreType.DMA,
    ],
)
def _sc_agg(table, srcp, dstp, out, sidx, didx, rows, acc, sem):
    c = lax.axis_index("c")
    s = lax.axis_index("s")
    tile = c * NS + s

    def zr(r, _):
        def zc(j, _2):
            rows[r, pl.ds(j * 16, 16)] = jnp.zeros((16,), _f32)
            return 0

        lax.fori_loop(0, DIM // 16, zc, 0)
        return 0

    lax.fori_loop(0, CH, zr, 0)

    def zi(k, _):
        pltpu.sync_copy(rows, acc.at[pl.ds(s * RPT + k * CH, CH)])
        return 0

    lax.fori_loop(0, RPT // CH, zi, 0)
    plsc.subcore_barrier()

    base = tile * EPT

    def body(i, _):
        off = base + i * CH
        pltpu.sync_copy(srcp.at[pl.ds(off, CH)], sidx)
        pltpu.sync_copy(dstp.at[pl.ds(off, CH)], didx)
        pltpu.async_copy(table.at[sidx], rows, sem).wait()
        pltpu.sync_copy(rows, acc.at[didx], add=True)
        return 0

    lax.fori_loop(0, CE, body, 0)
    plsc.subcore_barrier()

    def co(k, _):
        r0 = s * RPT + k * CH
        pltpu.sync_copy(acc.at[pl.ds(r0, CH)], rows)
        pltpu.sync_copy(rows, out.at[pl.ds(c * NP + r0, CH)])
        return 0

    lax.fori_loop(0, RPT // CH, co, 0)


@functools.partial(
    pl.kernel,
    out_type=jax.ShapeDtypeStruct((ELP, 16), _f32),
    mesh=_mesh,
    scratch_types=[
        pltpu.VMEM((CH,), jnp.int32),
        pltpu.VMEM((CH,), jnp.int32),
        pltpu.VMEM((CH, DIM), _f32),
        pltpu.VMEM((CH, DIM), _f32),
        pltpu.VMEM((CH, 16), _f32),
        pltpu.SemaphoreType.DMA,
    ],
)
def _sc_decode(z, eli0, eli1, out, idx0, idx1, r0, r1, rbuf, sem):
    c = lax.axis_index("c")
    s = lax.axis_index("s")
    tile = c * NS + s
    base = tile * ELT

    def body(i, _):
        off = base + i * CH
        pltpu.sync_copy(eli0.at[pl.ds(off, CH)], idx0)
        pltpu.sync_copy(eli1.at[pl.ds(off, CH)], idx1)
        pltpu.async_copy(z.at[idx0], r0, sem).wait()
        pltpu.async_copy(z.at[idx1], r1, sem).wait()

        def edge(e, _2):
            acc = r0[e, pl.ds(0, 16)] * r1[e, pl.ds(0, 16)]
            for j in range(1, DIM // 16):
                acc = acc + r0[e, pl.ds(j * 16, 16)] * r1[e, pl.ds(j * 16, 16)]
            rbuf[e, :] = acc
            return 0

        lax.fori_loop(0, CH, edge, 0)
        pltpu.sync_copy(rbuf, out.at[pl.ds(off, CH)])
        return 0

    lax.fori_loop(0, CD, body, 0)


_R = 512
_GRID = NP // _R


def _dot(a, b):
    return jnp.dot(a, b, preferred_element_type=_f32,
                   precision=lax.Precision.HIGHEST)


def _tc1_body(x_ref, w_ref, c0_ref, c1_ref, xwp_ref, d2_ref):
    deg = c0_ref[:, 0:1] + c1_ref[:, 0:1] + 1.0
    dinv = 1.0 / jnp.sqrt(deg)
    xwp_ref[...] = dinv * _dot(x_ref[...], w_ref[...])
    d2_ref[...] = jnp.broadcast_to(dinv, d2_ref.shape)


def _tc1(x_pad, W1, cnt2):
    return pl.pallas_call(
        _tc1_body,
        grid=(_GRID,),
        in_specs=[
            pl.BlockSpec((_R, DIM), lambda i: (i, 0)),
            pl.BlockSpec((DIM, DIM), lambda i: (0, 0)),
            pl.BlockSpec((_R, DIM), lambda i: (i, 0)),
            pl.BlockSpec((_R, DIM), lambda i: (i + _GRID, 0)),
        ],
        out_specs=[
            pl.BlockSpec((_R, DIM), lambda i: (i, 0)),
            pl.BlockSpec((_R, DIM), lambda i: (i, 0)),
        ],
        out_shape=[
            jax.ShapeDtypeStruct((NP, DIM), _f32),
            jax.ShapeDtypeStruct((NP, DIM), _f32),
        ],
    )(x_pad, W1, cnt2, cnt2)


def _tc_mid_body(agg0_ref, agg1_ref, xwp_ref, d2_ref, b_ref, w_ref, out_ref):
    d2 = d2_ref[...]
    pre = d2 * (agg0_ref[...] + agg1_ref[...] + xwp_ref[...]) + b_ref[...]
    h = jnp.maximum(pre, 0.0)
    out_ref[...] = d2 * _dot(h, w_ref[...])


def _tc_mid(agg, xwp, d2, brow, W):
    return pl.pallas_call(
        _tc_mid_body,
        grid=(_GRID,),
        in_specs=[
            pl.BlockSpec((_R, DIM), lambda i: (i, 0)),
            pl.BlockSpec((_R, DIM), lambda i: (i + _GRID, 0)),
            pl.BlockSpec((_R, DIM), lambda i: (i, 0)),
            pl.BlockSpec((_R, DIM), lambda i: (i, 0)),
            pl.BlockSpec((1, DIM), lambda i: (0, 0)),
            pl.BlockSpec((DIM, DIM), lambda i: (0, 0)),
        ],
        out_specs=pl.BlockSpec((_R, DIM), lambda i: (i, 0)),
        out_shape=jax.ShapeDtypeStruct((NP, DIM), _f32),
    )(agg, agg, xwp, d2, brow, W)


def _tc_fin_body(agg0_ref, agg1_ref, xwp_ref, d2_ref, b_ref, out_ref):
    out_ref[...] = (d2_ref[...] * (agg0_ref[...] + agg1_ref[...] + xwp_ref[...])
                    + b_ref[...])


def _tc_fin(agg, xwp, d2, brow):
    return pl.pallas_call(
        _tc_fin_body,
        grid=(_GRID,),
        in_specs=[
            pl.BlockSpec((_R, DIM), lambda i: (i, 0)),
            pl.BlockSpec((_R, DIM), lambda i: (i + _GRID, 0)),
            pl.BlockSpec((_R, DIM), lambda i: (i, 0)),
            pl.BlockSpec((_R, DIM), lambda i: (i, 0)),
            pl.BlockSpec((1, DIM), lambda i: (0, 0)),
        ],
        out_specs=pl.BlockSpec((_R, DIM), lambda i: (i, 0)),
        out_shape=jax.ShapeDtypeStruct((NP, DIM), _f32),
    )(agg, agg, xwp, d2, brow)


_RB = 2048


def _tc_lsum_body(r_ref, out_ref):
    out_ref[...] = jnp.sum(r_ref[...], axis=1)


def _tc_lsum(res16):
    return pl.pallas_call(
        _tc_lsum_body,
        grid=(ELP // _RB,),
        in_specs=[pl.BlockSpec((_RB, 16), lambda i: (i, 0))],
        out_specs=pl.BlockSpec((_RB,), lambda i: (i,)),
        out_shape=jax.ShapeDtypeStruct((ELP,), _f32),
    )(res16)


def kernel(x, edge_index, edge_label_index, W1, b1, W2, b2, W3, b3):
    i32 = jnp.int32
    src = edge_index[0].astype(i32)
    dst = edge_index[1].astype(i32)
    pad_e = jnp.full((EP - E,), N, i32)
    srcp = jnp.concatenate([src, pad_e])
    dstp = jnp.concatenate([dst, pad_e])
    pad_l = jnp.zeros((ELP - EL,), i32)
    eli0 = jnp.concatenate([edge_label_index[0].astype(i32), pad_l])
    eli1 = jnp.concatenate([edge_label_index[1].astype(i32), pad_l])
    x_pad = jnp.concatenate([x, jnp.zeros((NP - N, DIM), _f32)], axis=0)

    cnt2 = _sc_count(dstp)
    xw1p, d2 = _tc1(x_pad, W1, cnt2)
    agg1 = _sc_agg(xw1p, srcp, dstp)
    xw2p = _tc_mid(agg1, xw1p, d2, b1.reshape(1, DIM), W2)
    agg2 = _sc_agg(xw2p, srcp, dstp)
    xw3p = _tc_mid(agg2, xw2p, d2, b2.reshape(1, DIM), W3)
    agg3 = _sc_agg(xw3p, srcp, dstp)
    z = _tc_fin(agg3, xw3p, d2, b3.reshape(1, DIM))
    res16 = _sc_decode(z, eli0, eli1)
    dots = _tc_lsum(res16)
    return dots[:EL]


# speedup vs baseline: 1.8598x; 1.2114x over previous
import functools

import jax
import jax.numpy as jnp
from jax import lax
from jax.experimental import pallas as pl
from jax.experimental.pallas import tpu as pltpu
from jax.experimental.pallas import tpu_sc as plsc

N = 10000
E = 320000
EL = 100000
DIM = 128

NC = 2
NS = 16
NT = NC * NS

NP = 10240
RPT = NP // NS
CH = 128
EP = 323584
EPT = EP // NT
CE = EPT // CH
ELP = 102400
ELT = ELP // NT
CD = ELT // CH

_mesh = plsc.VectorSubcoreMesh(core_axis_name="c", subcore_axis_name="s")
_f32 = jnp.float32


def _fill16(buf, nrow, val):
    v = jnp.full((16,), val, _f32)

    def body(r, _):
        buf[r, :] = v
        return 0

    lax.fori_loop(0, nrow, body, 0)


@functools.partial(
    pl.kernel,
    out_type=jax.ShapeDtypeStruct((2 * NP, DIM), _f32),
    mesh=_mesh,
    scratch_types=[
        pltpu.VMEM((CE, CH), jnp.int32),
        pltpu.VMEM((CH, DIM), _f32),
        pltpu.VMEM_SHARED((NP, DIM), _f32),
    ],
)
def _sc_count(dstp3, out, didx, rows, acc):
    c = lax.axis_index("c")
    s = lax.axis_index("s")
    tile = c * NS + s
    pltpu.sync_copy(dstp3.at[tile], didx)

    def zr(r, _):
        def zc(j, _2):
            rows[r, pl.ds(j * 16, 16)] = jnp.zeros((16,), _f32)
            return 0

        lax.fori_loop(0, DIM // 16, zc, 0)
        return 0

    lax.fori_loop(0, CH, zr, 0)

    def zi(k, _):
        pltpu.sync_copy(rows, acc.at[pl.ds(s * RPT + k * CH, CH)])
        return 0

    lax.fori_loop(0, RPT // CH, zi, 0)

    def fr(r, _):
        def fc(j, _2):
            rows[r, pl.ds(j * 16, 16)] = jnp.full((16,), 1.0, _f32)
            return 0

        lax.fori_loop(0, DIM // 16, fc, 0)
        return 0

    lax.fori_loop(0, CH, fr, 0)
    plsc.subcore_barrier()

    def body(i, _):
        pltpu.sync_copy(rows, acc.at[didx.at[i]], add=True)
        return 0

    lax.fori_loop(0, CE, body, 0)
    plsc.subcore_barrier()

    def co(k, _):
        r0 = s * RPT + k * CH
        pltpu.sync_copy(acc.at[pl.ds(r0, CH)], rows)
        pltpu.sync_copy(rows, out.at[pl.ds(c * NP + r0, CH)])
        return 0

    lax.fori_loop(0, RPT // CH, co, 0)


@functools.partial(
    pl.kernel,
    out_type=jax.ShapeDtypeStruct((2 * NP, DIM), _f32),
    mesh=_mesh,
    scratch_types=[
        pltpu.VMEM((CE, CH), jnp.int32),
        pltpu.VMEM((CE, CH), jnp.int32),
        pltpu.VMEM((CH, DIM), _f32),
        pltpu.VMEM_SHARED((NP, DIM), _f32),
        pltpu.SemaphoreType.DMA,
    ],
)
def _sc_agg(table, srcp3, dstp3, out, sidx, didx, rows, acc, sem):
    c = lax.axis_index("c")
    s = lax.axis_index("s")
    tile = c * NS + s
    pltpu.sync_copy(srcp3.at[tile], sidx)
    pltpu.sync_copy(dstp3.at[tile], didx)

    def zr(r, _):
        def zc(j, _2):
            rows[r, pl.ds(j * 16, 16)] = jnp.zeros((16,), _f32)
            return 0

        lax.fori_loop(0, DIM // 16, zc, 0)
        return 0

    lax.fori_loop(0, CH, zr, 0)

    def zi(k, _):
        pltpu.sync_copy(rows, acc.at[pl.ds(s * RPT + k * CH, CH)])
        return 0

    lax.fori_loop(0, RPT // CH, zi, 0)
    plsc.subcore_barrier()

    def body(i, _):
        pltpu.async_copy(table.at[sidx.at[i]], rows, sem).wait()
        pltpu.sync_copy(rows, acc.at[didx.at[i]], add=True)
        return 0

    lax.fori_loop(0, CE, body, 0)
    plsc.subcore_barrier()

    def co(k, _):
        r0 = s * RPT + k * CH
        pltpu.sync_copy(acc.at[pl.ds(r0, CH)], rows)
        pltpu.sync_copy(rows, out.at[pl.ds(c * NP + r0, CH)])
        return 0

    lax.fori_loop(0, RPT // CH, co, 0)


@functools.partial(
    pl.kernel,
    out_type=jax.ShapeDtypeStruct((ELP, 16), _f32),
    mesh=_mesh,
    scratch_types=[
        pltpu.VMEM((CD, CH), jnp.int32),
        pltpu.VMEM((CD, CH), jnp.int32),
        pltpu.VMEM((CH, DIM), _f32),
        pltpu.VMEM((CH, DIM), _f32),
        pltpu.VMEM((CH, 16), _f32),
        pltpu.SemaphoreType.DMA,
    ],
)
def _sc_decode(z, eli03, eli13, out, idx0, idx1, r0, r1, rbuf, sem):
    c = lax.axis_index("c")
    s = lax.axis_index("s")
    tile = c * NS + s
    pltpu.sync_copy(eli03.at[tile], idx0)
    pltpu.sync_copy(eli13.at[tile], idx1)
    base = tile * ELT

    def body(i, _):
        off = base + i * CH
        pltpu.async_copy(z.at[idx0.at[i]], r0, sem).wait()
        pltpu.async_copy(z.at[idx1.at[i]], r1, sem).wait()

        def edge(e, _2):
            acc = r0[e, pl.ds(0, 16)] * r1[e, pl.ds(0, 16)]
            for j in range(1, DIM // 16):
                acc = acc + r0[e, pl.ds(j * 16, 16)] * r1[e, pl.ds(j * 16, 16)]
            rbuf[e, :] = acc
            return 0

        lax.fori_loop(0, CH, edge, 0)
        pltpu.sync_copy(rbuf, out.at[pl.ds(off, CH)])
        return 0

    lax.fori_loop(0, CD, body, 0)


_R = 512
_GRID = NP // _R


def _dot(a, b):
    return jnp.dot(a, b, preferred_element_type=_f32,
                   precision=lax.Precision.HIGHEST)


def _tc1_body(x_ref, w_ref, c0_ref, c1_ref, xwp_ref, d2_ref):
    deg = c0_ref[:, 0:1] + c1_ref[:, 0:1] + 1.0
    dinv = 1.0 / jnp.sqrt(deg)
    xwp_ref[...] = dinv * _dot(x_ref[...], w_ref[...])
    d2_ref[...] = jnp.broadcast_to(dinv, d2_ref.shape)


def _tc1(x_pad, W1, cnt2):
    return pl.pallas_call(
        _tc1_body,
        grid=(_GRID,),
        in_specs=[
            pl.BlockSpec((_R, DIM), lambda i: (i, 0)),
            pl.BlockSpec((DIM, DIM), lambda i: (0, 0)),
            pl.BlockSpec((_R, DIM), lambda i: (i, 0)),
            pl.BlockSpec((_R, DIM), lambda i: (i + _GRID, 0)),
        ],
        out_specs=[
            pl.BlockSpec((_R, DIM), lambda i: (i, 0)),
            pl.BlockSpec((_R, DIM), lambda i: (i, 0)),
        ],
        out_shape=[
            jax.ShapeDtypeStruct((NP, DIM), _f32),
            jax.ShapeDtypeStruct((NP, DIM), _f32),
        ],
    )(x_pad, W1, cnt2, cnt2)


def _tc_mid_body(agg0_ref, agg1_ref, xwp_ref, d2_ref, b_ref, w_ref, out_ref):
    d2 = d2_ref[...]
    pre = d2 * (agg0_ref[...] + agg1_ref[...] + xwp_ref[...]) + b_ref[...]
    h = jnp.maximum(pre, 0.0)
    out_ref[...] = d2 * _dot(h, w_ref[...])


def _tc_mid(agg, xwp, d2, brow, W):
    return pl.pallas_call(
        _tc_mid_body,
        grid=(_GRID,),
        in_specs=[
            pl.BlockSpec((_R, DIM), lambda i: (i, 0)),
            pl.BlockSpec((_R, DIM), lambda i: (i + _GRID, 0)),
            pl.BlockSpec((_R, DIM), lambda i: (i, 0)),
            pl.BlockSpec((_R, DIM), lambda i: (i, 0)),
            pl.BlockSpec((1, DIM), lambda i: (0, 0)),
            pl.BlockSpec((DIM, DIM), lambda i: (0, 0)),
        ],
        out_specs=pl.BlockSpec((_R, DIM), lambda i: (i, 0)),
        out_shape=jax.ShapeDtypeStruct((NP, DIM), _f32),
    )(agg, agg, xwp, d2, brow, W)


def _tc_fin_body(agg0_ref, agg1_ref, xwp_ref, d2_ref, b_ref, out_ref):
    out_ref[...] = (d2_ref[...] * (agg0_ref[...] + agg1_ref[...] + xwp_ref[...])
                    + b_ref[...])


def _tc_fin(agg, xwp, d2, brow):
    return pl.pallas_call(
        _tc_fin_body,
        grid=(_GRID,),
        in_specs=[
            pl.BlockSpec((_R, DIM), lambda i: (i, 0)),
            pl.BlockSpec((_R, DIM), lambda i: (i + _GRID, 0)),
            pl.BlockSpec((_R, DIM), lambda i: (i, 0)),
            pl.BlockSpec((_R, DIM), lambda i: (i, 0)),
            pl.BlockSpec((1, DIM), lambda i: (0, 0)),
        ],
        out_specs=pl.BlockSpec((_R, DIM), lambda i: (i, 0)),
        out_shape=jax.ShapeDtypeStruct((NP, DIM), _f32),
    )(agg, agg, xwp, d2, brow)


_RB = 2048


def _tc_lsum_body(r_ref, out_ref):
    out_ref[...] = jnp.sum(r_ref[...], axis=1)


def _tc_lsum(res16):
    return pl.pallas_call(
        _tc_lsum_body,
        grid=(ELP // _RB,),
        in_specs=[pl.BlockSpec((_RB, 16), lambda i: (i, 0))],
        out_specs=pl.BlockSpec((_RB,), lambda i: (i,)),
        out_shape=jax.ShapeDtypeStruct((ELP,), _f32),
    )(res16)


def kernel(x, edge_index, edge_label_index, W1, b1, W2, b2, W3, b3):
    i32 = jnp.int32
    src = edge_index[0].astype(i32)
    dst = edge_index[1].astype(i32)
    pad_e = jnp.full((EP - E,), N, i32)
    srcp3 = jnp.concatenate([src, pad_e]).reshape(NT, CE, CH)
    dstp3 = jnp.concatenate([dst, pad_e]).reshape(NT, CE, CH)
    pad_l = jnp.zeros((ELP - EL,), i32)
    eli03 = jnp.concatenate([edge_label_index[0].astype(i32),
                             pad_l]).reshape(NT, CD, CH)
    eli13 = jnp.concatenate([edge_label_index[1].astype(i32),
                             pad_l]).reshape(NT, CD, CH)
    x_pad = jnp.concatenate([x, jnp.zeros((NP - N, DIM), _f32)], axis=0)

    cnt2 = _sc_count(dstp3)
    xw1p, d2 = _tc1(x_pad, W1, cnt2)
    agg1 = _sc_agg(xw1p, srcp3, dstp3)
    xw2p = _tc_mid(agg1, xw1p, d2, b1.reshape(1, DIM), W2)
    agg2 = _sc_agg(xw2p, srcp3, dstp3)
    xw3p = _tc_mid(agg2, xw2p, d2, b2.reshape(1, DIM), W3)
    agg3 = _sc_agg(xw3p, srcp3, dstp3)
    z = _tc_fin(agg3, xw3p, d2, b3.reshape(1, DIM))
    res16 = _sc_decode(z, eli03, eli13)
    dots = _tc_lsum(res16)
    return dots[:EL]


# speedup vs baseline: 2.4986x; 1.3435x over previous
import functools

import jax
import jax.numpy as jnp
from jax import lax
from jax.experimental import pallas as pl
from jax.experimental.pallas import tpu as pltpu
from jax.experimental.pallas import tpu_sc as plsc

N = 10000
E = 320000
EL = 100000
DIM = 128

NC = 2
NS = 16
NT = NC * NS

NP = 10240
RPT = NP // NS
CH = 128
EP = 323584
EPT = EP // NT
CE = EPT // CH
ELP = 102400
ELT = ELP // NT
CD = ELT // CH

_mesh = plsc.VectorSubcoreMesh(core_axis_name="c", subcore_axis_name="s")
_f32 = jnp.float32


def _fill16(buf, nrow, val):
    v = jnp.full((16,), val, _f32)

    def body(r, _):
        buf[r, :] = v
        return 0

    lax.fori_loop(0, nrow, body, 0)


@functools.partial(
    pl.kernel,
    out_type=jax.ShapeDtypeStruct((2 * NP, DIM), _f32),
    mesh=_mesh,
    scratch_types=[
        pltpu.VMEM((CE, CH), jnp.int32),
        pltpu.VMEM((CH, DIM), _f32),
        pltpu.VMEM_SHARED((NP, DIM), _f32),
    ],
)
def _sc_count(dstp3, out, didx, rows, acc):
    c = lax.axis_index("c")
    s = lax.axis_index("s")
    tile = c * NS + s
    pltpu.sync_copy(dstp3.at[tile], didx)

    def zr(r, _):
        def zc(j, _2):
            rows[r, pl.ds(j * 16, 16)] = jnp.zeros((16,), _f32)
            return 0

        lax.fori_loop(0, DIM // 16, zc, 0)
        return 0

    lax.fori_loop(0, CH, zr, 0)

    def zi(k, _):
        pltpu.sync_copy(rows, acc.at[pl.ds(s * RPT + k * CH, CH)])
        return 0

    lax.fori_loop(0, RPT // CH, zi, 0)

    def fr(r, _):
        def fc(j, _2):
            rows[r, pl.ds(j * 16, 16)] = jnp.full((16,), 1.0, _f32)
            return 0

        lax.fori_loop(0, DIM // 16, fc, 0)
        return 0

    lax.fori_loop(0, CH, fr, 0)
    plsc.subcore_barrier()

    def body(i, _):
        pltpu.sync_copy(rows, acc.at[didx.at[i]], add=True)
        return 0

    lax.fori_loop(0, CE, body, 0)
    plsc.subcore_barrier()

    def co(k, _):
        r0 = s * RPT + k * CH
        pltpu.sync_copy(acc.at[pl.ds(r0, CH)], rows)
        pltpu.sync_copy(rows, out.at[pl.ds(c * NP + r0, CH)])
        return 0

    lax.fori_loop(0, RPT // CH, co, 0)


@functools.partial(
    pl.kernel,
    out_type=jax.ShapeDtypeStruct((2 * NP, DIM), _f32),
    mesh=_mesh,
    scratch_types=[
        pltpu.VMEM((CE, CH), jnp.int32),
        pltpu.VMEM((CE, CH), jnp.int32),
        pltpu.VMEM((CH, DIM), _f32),
        pltpu.VMEM_SHARED((NP, DIM), _f32),
        pltpu.SemaphoreType.DMA,
    ],
)
def _sc_agg(table, srcp3, dstp3, out, sidx, didx, rows, acc, sem):
    c = lax.axis_index("c")
    s = lax.axis_index("s")
    tile = c * NS + s
    pltpu.sync_copy(srcp3.at[tile], sidx)
    pltpu.sync_copy(dstp3.at[tile], didx)

    def zr(r, _):
        def zc(j, _2):
            rows[r, pl.ds(j * 16, 16)] = jnp.zeros((16,), _f32)
            return 0

        lax.fori_loop(0, DIM // 16, zc, 0)
        return 0

    lax.fori_loop(0, CH, zr, 0)

    def zi(k, _):
        pltpu.sync_copy(rows, acc.at[pl.ds(s * RPT + k * CH, CH)])
        return 0

    lax.fori_loop(0, RPT // CH, zi, 0)
    plsc.subcore_barrier()

    def body(i, _):
        pltpu.async_copy(table.at[sidx.at[i]], rows, sem).wait()
        pltpu.sync_copy(rows, acc.at[didx.at[i]], add=True)
        return 0

    lax.fori_loop(0, CE, body, 0)
    plsc.subcore_barrier()

    def co(k, _):
        r0 = s * RPT + k * CH
        pltpu.sync_copy(acc.at[pl.ds(r0, CH)], rows)
        pltpu.sync_copy(rows, out.at[pl.ds(c * NP + r0, CH)])
        return 0

    lax.fori_loop(0, RPT // CH, co, 0)


@functools.partial(
    pl.kernel,
    out_type=jax.ShapeDtypeStruct((ELP, 16), _f32),
    mesh=_mesh,
    scratch_types=[
        pltpu.VMEM((CD, CH), jnp.int32),
        pltpu.VMEM((CD, CH), jnp.int32),
        pltpu.VMEM((CH, DIM), _f32),
        pltpu.VMEM((CH, DIM), _f32),
        pltpu.VMEM((CH, 16), _f32),
        pltpu.SemaphoreType.DMA,
    ],
)
def _sc_decode(z, eli03, eli13, out, idx0, idx1, r0, r1, rbuf, sem):
    c = lax.axis_index("c")
    s = lax.axis_index("s")
    tile = c * NS + s
    pltpu.sync_copy(eli03.at[tile], idx0)
    pltpu.sync_copy(eli13.at[tile], idx1)
    base = tile * ELT

    def body(i, _):
        off = base + i * CH
        pltpu.async_copy(z.at[idx0.at[i]], r0, sem).wait()
        pltpu.async_copy(z.at[idx1.at[i]], r1, sem).wait()

        def edge(e, _2):
            acc = r0[e, pl.ds(0, 16)] * r1[e, pl.ds(0, 16)]
            for j in range(1, DIM // 16):
                acc = acc + r0[e, pl.ds(j * 16, 16)] * r1[e, pl.ds(j * 16, 16)]
            rbuf[e, :] = acc
            return 0

        lax.fori_loop(0, CH, edge, 0)
        pltpu.sync_copy(rbuf, out.at[pl.ds(off, CH)])
        return 0

    lax.fori_loop(0, CD, body, 0)


_R = 512
_GRID = NP // _R


def _dot(a, b):
    return jnp.dot(a, b, preferred_element_type=_f32,
                   precision=lax.Precision.HIGHEST)


def _tc1_body(x_ref, w_ref, c0_ref, c1_ref, xwp_ref, d2_ref):
    deg = c0_ref[:, 0:1] + c1_ref[:, 0:1] + 1.0
    dinv = 1.0 / jnp.sqrt(deg)
    xwp_ref[...] = dinv * _dot(x_ref[...], w_ref[...])
    d2_ref[...] = jnp.broadcast_to(dinv, d2_ref.shape)


def _tc1(x_pad, W1, cnt2):
    return pl.pallas_call(
        _tc1_body,
        grid=(_GRID,),
        in_specs=[
            pl.BlockSpec((_R, DIM), lambda i: (i, 0)),
            pl.BlockSpec((DIM, DIM), lambda i: (0, 0)),
            pl.BlockSpec((_R, DIM), lambda i: (i, 0)),
            pl.BlockSpec((_R, DIM), lambda i: (i + _GRID, 0)),
        ],
        out_specs=[
            pl.BlockSpec((_R, DIM), lambda i: (i, 0)),
            pl.BlockSpec((_R, DIM), lambda i: (i, 0)),
        ],
        out_shape=[
            jax.ShapeDtypeStruct((NP, DIM), _f32),
            jax.ShapeDtypeStruct((NP, DIM), _f32),
        ],
    )(x_pad, W1, cnt2, cnt2)


def _tc_mid_body(agg0_ref, agg1_ref, xwp_ref, d2_ref, b_ref, w_ref, out_ref):
    d2 = d2_ref[...]
    pre = d2 * (agg0_ref[...] + agg1_ref[...] + xwp_ref[...]) + b_ref[...]
    h = jnp.maximum(pre, 0.0)
    out_ref[...] = d2 * _dot(h, w_ref[...])


def _tc_mid(agg, xwp, d2, brow, W):
    return pl.pallas_call(
        _tc_mid_body,
        grid=(_GRID,),
        in_specs=[
            pl.BlockSpec((_R, DIM), lambda i: (i, 0)),
            pl.BlockSpec((_R, DIM), lambda i: (i + _GRID, 0)),
            pl.BlockSpec((_R, DIM), lambda i: (i, 0)),
            pl.BlockSpec((_R, DIM), lambda i: (i, 0)),
            pl.BlockSpec((1, DIM), lambda i: (0, 0)),
            pl.BlockSpec((DIM, DIM), lambda i: (0, 0)),
        ],
        out_specs=pl.BlockSpec((_R, DIM), lambda i: (i, 0)),
        out_shape=jax.ShapeDtypeStruct((NP, DIM), _f32),
    )(agg, agg, xwp, d2, brow, W)


def _tc_fin_body(agg0_ref, agg1_ref, xwp_ref, d2_ref, b_ref, out_ref):
    out_ref[...] = (d2_ref[...] * (agg0_ref[...] + agg1_ref[...] + xwp_ref[...])
                    + b_ref[...])


def _tc_fin(agg, xwp, d2, brow):
    return pl.pallas_call(
        _tc_fin_body,
        grid=(_GRID,),
        in_specs=[
            pl.BlockSpec((_R, DIM), lambda i: (i, 0)),
            pl.BlockSpec((_R, DIM), lambda i: (i + _GRID, 0)),
            pl.BlockSpec((_R, DIM), lambda i: (i, 0)),
            pl.BlockSpec((_R, DIM), lambda i: (i, 0)),
            pl.BlockSpec((1, DIM), lambda i: (0, 0)),
        ],
        out_specs=pl.BlockSpec((_R, DIM), lambda i: (i, 0)),
        out_shape=jax.ShapeDtypeStruct((NP, DIM), _f32),
    )(agg, agg, xwp, d2, brow)


_RB = 2048


def _tc_lsum_body(r_ref, out_ref):
    out_ref[...] = jnp.sum(r_ref[...], axis=1)


def _tc_lsum(res16):
    return pl.pallas_call(
        _tc_lsum_body,
        grid=(ELP // _RB,),
        in_specs=[pl.BlockSpec((_RB, 16), lambda i: (i, 0))],
        out_specs=pl.BlockSpec((_RB,), lambda i: (i,)),
        out_shape=jax.ShapeDtypeStruct((ELP,), _f32),
    )(res16)


def kernel(x, edge_index, edge_label_index, W1, b1, W2, b2, W3, b3):
    i32 = jnp.int32
    src = edge_index[0].astype(i32)
    dst = edge_index[1].astype(i32)
    pad_e = N + jnp.arange(EP - E, dtype=i32) % (NP - N)
    srcp3 = jnp.concatenate([src, pad_e]).reshape(NT, CE, CH)
    dstp3 = jnp.concatenate([dst, pad_e]).reshape(NT, CE, CH)
    pad_l = jnp.zeros((ELP - EL,), i32)
    eli03 = jnp.concatenate([edge_label_index[0].astype(i32),
                             pad_l]).reshape(NT, CD, CH)
    eli13 = jnp.concatenate([edge_label_index[1].astype(i32),
                             pad_l]).reshape(NT, CD, CH)
    x_pad = jnp.concatenate([x, jnp.zeros((NP - N, DIM), _f32)], axis=0)

    cnt2 = _sc_count(dstp3)
    xw1p, d2 = _tc1(x_pad, W1, cnt2)
    agg1 = _sc_agg(xw1p, srcp3, dstp3)
    xw2p = _tc_mid(agg1, xw1p, d2, b1.reshape(1, DIM), W2)
    agg2 = _sc_agg(xw2p, srcp3, dstp3)
    xw3p = _tc_mid(agg2, xw2p, d2, b2.reshape(1, DIM), W3)
    agg3 = _sc_agg(xw3p, srcp3, dstp3)
    z = _tc_fin(agg3, xw3p, d2, b3.reshape(1, DIM))
    res16 = _sc_decode(z, eli03, eli13)
    dots = _tc_lsum(res16)
    return dots[:EL]


# speedup vs baseline: 2.5187x; 1.0080x over previous
import functools

import jax
import jax.numpy as jnp
from jax import lax
from jax.experimental import pallas as pl
from jax.experimental.pallas import tpu as pltpu
from jax.experimental.pallas import tpu_sc as plsc

N = 10000
E = 320000
EL = 100000
DIM = 128

NC = 2
NS = 16
NT = NC * NS

NP = 10240
RPT = NP // NS
CH = 128
EP = 323584
EPT = EP // NT
CE = EPT // CH
ELP = 102400
ELT = ELP // NT
CD = ELT // CH

_mesh = plsc.VectorSubcoreMesh(core_axis_name="c", subcore_axis_name="s")
_f32 = jnp.float32


def _fill16(buf, nrow, val):
    v = jnp.full((16,), val, _f32)

    def body(r, _):
        buf[r, :] = v
        return 0

    lax.fori_loop(0, nrow, body, 0)


@functools.partial(
    pl.kernel,
    out_type=jax.ShapeDtypeStruct((2 * NP, DIM), _f32),
    mesh=_mesh,
    scratch_types=[
        pltpu.VMEM((CE, CH), jnp.int32),
        pltpu.VMEM((CH, DIM), _f32),
        pltpu.VMEM_SHARED((NP, DIM), _f32),
    ],
)
def _sc_count(dstp3, out, didx, rows, acc):
    c = lax.axis_index("c")
    s = lax.axis_index("s")
    tile = c * NS + s
    pltpu.sync_copy(dstp3.at[tile], didx)

    def zr(r, _):
        def zc(j, _2):
            rows[r, pl.ds(j * 16, 16)] = jnp.zeros((16,), _f32)
            return 0

        lax.fori_loop(0, DIM // 16, zc, 0)
        return 0

    lax.fori_loop(0, CH, zr, 0)

    def zi(k, _):
        pltpu.sync_copy(rows, acc.at[pl.ds(s * RPT + k * CH, CH)])
        return 0

    lax.fori_loop(0, RPT // CH, zi, 0)

    def fr(r, _):
        def fc(j, _2):
            rows[r, pl.ds(j * 16, 16)] = jnp.full((16,), 1.0, _f32)
            return 0

        lax.fori_loop(0, DIM // 16, fc, 0)
        return 0

    lax.fori_loop(0, CH, fr, 0)
    plsc.subcore_barrier()

    def body(i, _):
        pltpu.sync_copy(rows, acc.at[didx.at[i]], add=True)
        return 0

    lax.fori_loop(0, CE, body, 0)
    plsc.subcore_barrier()

    def co(k, _):
        r0 = s * RPT + k * CH
        pltpu.sync_copy(acc.at[pl.ds(r0, CH)], rows)
        pltpu.sync_copy(rows, out.at[pl.ds(c * NP + r0, CH)])
        return 0

    lax.fori_loop(0, RPT // CH, co, 0)


@functools.partial(
    pl.kernel,
    out_type=jax.ShapeDtypeStruct((2 * NP, DIM), _f32),
    mesh=_mesh,
    scratch_types=[
        pltpu.VMEM((CE, CH), jnp.int32),
        pltpu.VMEM((CE, CH), jnp.int32),
        pltpu.VMEM((CH, DIM), _f32),
        pltpu.VMEM_SHARED((NP, DIM), _f32),
        pltpu.SemaphoreType.DMA,
    ],
)
def _sc_agg(table, srcp3, dstp3, out, sidx, didx, rows, acc, sem):
    c = lax.axis_index("c")
    s = lax.axis_index("s")
    tile = c * NS + s
    pltpu.sync_copy(srcp3.at[tile], sidx)
    pltpu.sync_copy(dstp3.at[tile], didx)

    def zr(r, _):
        def zc(j, _2):
            rows[r, pl.ds(j * 16, 16)] = jnp.zeros((16,), _f32)
            return 0

        lax.fori_loop(0, DIM // 16, zc, 0)
        return 0

    lax.fori_loop(0, CH, zr, 0)

    def zi(k, _):
        pltpu.sync_copy(rows, acc.at[pl.ds(s * RPT + k * CH, CH)])
        return 0

    lax.fori_loop(0, RPT // CH, zi, 0)
    plsc.subcore_barrier()

    def body(i, _):
        pltpu.async_copy(table.at[sidx.at[i]], rows, sem).wait()
        pltpu.sync_copy(rows, acc.at[didx.at[i]], add=True)
        return 0

    lax.fori_loop(0, CE, body, 0)
    plsc.subcore_barrier()

    def co(k, _):
        r0 = s * RPT + k * CH
        pltpu.sync_copy(acc.at[pl.ds(r0, CH)], rows)
        pltpu.sync_copy(rows, out.at[pl.ds(c * NP + r0, CH)])
        return 0

    lax.fori_loop(0, RPT // CH, co, 0)


@functools.partial(
    pl.kernel,
    out_type=jax.ShapeDtypeStruct((ELP, 16), _f32),
    mesh=_mesh,
    scratch_types=[
        pltpu.VMEM((CD, CH), jnp.int32),
        pltpu.VMEM((CD, CH), jnp.int32),
        pltpu.VMEM((CH, DIM), _f32),
        pltpu.VMEM((CH, DIM), _f32),
        pltpu.VMEM((CH, 16), _f32),
        pltpu.SemaphoreType.DMA,
        pltpu.SemaphoreType.DMA,
    ],
)
def _sc_decode(z, eli03, eli13, out, idx0, idx1, r0, r1, rbuf, sem, sem1):
    c = lax.axis_index("c")
    s = lax.axis_index("s")
    tile = c * NS + s
    pltpu.sync_copy(eli03.at[tile], idx0)
    pltpu.sync_copy(eli13.at[tile], idx1)
    base = tile * ELT

    def body(i, _):
        off = base + i * CH
        h0 = pltpu.async_copy(z.at[idx0.at[i]], r0, sem)
        h1 = pltpu.async_copy(z.at[idx1.at[i]], r1, sem1)
        h0.wait()
        h1.wait()

        def edge(e, _2):
            acc = r0[e, pl.ds(0, 16)] * r1[e, pl.ds(0, 16)]
            for j in range(1, DIM // 16):
                acc = acc + r0[e, pl.ds(j * 16, 16)] * r1[e, pl.ds(j * 16, 16)]
            rbuf[e, :] = acc
            return 0

        lax.fori_loop(0, CH, edge, 0)
        pltpu.sync_copy(rbuf, out.at[pl.ds(off, CH)])
        return 0

    lax.fori_loop(0, CD, body, 0)


_R = 512
_GRID = NP // _R


def _dot(a, b):
    return jnp.dot(a, b, preferred_element_type=_f32,
                   precision=lax.Precision.HIGHEST)


def _tc1_body(x_ref, w_ref, c0_ref, c1_ref, xwp_ref, d2_ref):
    deg = c0_ref[:, 0:1] + c1_ref[:, 0:1] + 1.0
    dinv = 1.0 / jnp.sqrt(deg)
    xwp_ref[...] = dinv * _dot(x_ref[...], w_ref[...])
    d2_ref[...] = jnp.broadcast_to(dinv, d2_ref.shape)


def _tc1(x_pad, W1, cnt2):
    return pl.pallas_call(
        _tc1_body,
        grid=(_GRID,),
        in_specs=[
            pl.BlockSpec((_R, DIM), lambda i: (i, 0)),
            pl.BlockSpec((DIM, DIM), lambda i: (0, 0)),
            pl.BlockSpec((_R, DIM), lambda i: (i, 0)),
            pl.BlockSpec((_R, DIM), lambda i: (i + _GRID, 0)),
        ],
        out_specs=[
            pl.BlockSpec((_R, DIM), lambda i: (i, 0)),
            pl.BlockSpec((_R, DIM), lambda i: (i, 0)),
        ],
        out_shape=[
            jax.ShapeDtypeStruct((NP, DIM), _f32),
            jax.ShapeDtypeStruct((NP, DIM), _f32),
        ],
    )(x_pad, W1, cnt2, cnt2)


def _tc_mid_body(agg0_ref, agg1_ref, xwp_ref, d2_ref, b_ref, w_ref, out_ref):
    d2 = d2_ref[...]
    pre = d2 * (agg0_ref[...] + agg1_ref[...] + xwp_ref[...]) + b_ref[...]
    h = jnp.maximum(pre, 0.0)
    out_ref[...] = d2 * _dot(h, w_ref[...])


def _tc_mid(agg, xwp, d2, brow, W):
    return pl.pallas_call(
        _tc_mid_body,
        grid=(_GRID,),
        in_specs=[
            pl.BlockSpec((_R, DIM), lambda i: (i, 0)),
            pl.BlockSpec((_R, DIM), lambda i: (i + _GRID, 0)),
            pl.BlockSpec((_R, DIM), lambda i: (i, 0)),
            pl.BlockSpec((_R, DIM), lambda i: (i, 0)),
            pl.BlockSpec((1, DIM), lambda i: (0, 0)),
            pl.BlockSpec((DIM, DIM), lambda i: (0, 0)),
        ],
        out_specs=pl.BlockSpec((_R, DIM), lambda i: (i, 0)),
        out_shape=jax.ShapeDtypeStruct((NP, DIM), _f32),
    )(agg, agg, xwp, d2, brow, W)


def _tc_fin_body(agg0_ref, agg1_ref, xwp_ref, d2_ref, b_ref, out_ref):
    out_ref[...] = (d2_ref[...] * (agg0_ref[...] + agg1_ref[...] + xwp_ref[...])
                    + b_ref[...])


def _tc_fin(agg, xwp, d2, brow):
    return pl.pallas_call(
        _tc_fin_body,
        grid=(_GRID,),
        in_specs=[
            pl.BlockSpec((_R, DIM), lambda i: (i, 0)),
            pl.BlockSpec((_R, DIM), lambda i: (i + _GRID, 0)),
            pl.BlockSpec((_R, DIM), lambda i: (i, 0)),
            pl.BlockSpec((_R, DIM), lambda i: (i, 0)),
            pl.BlockSpec((1, DIM), lambda i: (0, 0)),
        ],
        out_specs=pl.BlockSpec((_R, DIM), lambda i: (i, 0)),
        out_shape=jax.ShapeDtypeStruct((NP, DIM), _f32),
    )(agg, agg, xwp, d2, brow)


_RB = 2048


def _tc_lsum_body(r_ref, out_ref):
    out_ref[...] = jnp.sum(r_ref[...], axis=1)


def _tc_lsum(res16):
    return pl.pallas_call(
        _tc_lsum_body,
        grid=(ELP // _RB,),
        in_specs=[pl.BlockSpec((_RB, 16), lambda i: (i, 0))],
        out_specs=pl.BlockSpec((_RB,), lambda i: (i,)),
        out_shape=jax.ShapeDtypeStruct((ELP,), _f32),
    )(res16)


def kernel(x, edge_index, edge_label_index, W1, b1, W2, b2, W3, b3):
    i32 = jnp.int32
    src = edge_index[0].astype(i32)
    dst = edge_index[1].astype(i32)
    pad_e = N + jnp.arange(EP - E, dtype=i32) % (NP - N)
    srcp3 = jnp.concatenate([src, pad_e]).reshape(NT, CE, CH)
    dstp3 = jnp.concatenate([dst, pad_e]).reshape(NT, CE, CH)
    pad_l = jnp.zeros((ELP - EL,), i32)
    eli03 = jnp.concatenate([edge_label_index[0].astype(i32),
                             pad_l]).reshape(NT, CD, CH)
    eli13 = jnp.concatenate([edge_label_index[1].astype(i32),
                             pad_l]).reshape(NT, CD, CH)
    x_pad = jnp.concatenate([x, jnp.zeros((NP - N, DIM), _f32)], axis=0)

    cnt2 = _sc_count(dstp3)
    xw1p, d2 = _tc1(x_pad, W1, cnt2)
    agg1 = _sc_agg(xw1p, srcp3, dstp3)
    xw2p = _tc_mid(agg1, xw1p, d2, b1.reshape(1, DIM), W2)
    agg2 = _sc_agg(xw2p, srcp3, dstp3)
    xw3p = _tc_mid(agg2, xw2p, d2, b2.reshape(1, DIM), W3)
    agg3 = _sc_agg(xw3p, srcp3, dstp3)
    z = _tc_fin(agg3, xw3p, d2, b3.reshape(1, DIM))
    res16 = _sc_decode(z, eli03, eli13)
    dots = _tc_lsum(res16)
    return dots[:EL]
